# Initial kernel scaffold; baseline (speedup 1.0000x reference)
import jax
import jax.numpy as jnp
from jax.experimental import pallas as pl


def kernel(hidden_states, router_w, w1, v1, w2):
    raise NotImplementedError("write your pallas kernel here")


# speedup vs baseline: 1.6859x; 1.6859x over previous
import functools

import jax
import jax.numpy as jnp
from jax import lax
from jax.experimental import pallas as pl
from jax.experimental.pallas import tpu as pltpu
from jax.experimental.pallas import tpu_sc as plsc

E = 8
TK = 2
D = 2048
F = 768
T = 2048
P = T * TK
BM = 256
NB = P // BM + E
NQ = NB * BM

NW = 32
G_CH = 32
C_CH = 16


def _router_sched_body(x_ref, rw_ref, pos_ref, topw_ref, meta_ref):
    x = x_ref[...]
    rw = rw_ref[...]
    lt = lax.dot_general(rw, x, (((1,), (1,)), ((), ())),
                         preferred_element_type=jnp.float32)
    iota_e = lax.broadcasted_iota(jnp.int32, (E, T), 0)
    l1 = jnp.max(lt, axis=0, keepdims=True)
    i1 = jnp.min(jnp.where(lt == l1, iota_e, E), axis=0, keepdims=True)
    m0 = (iota_e == i1)
    ltm = jnp.where(m0, -jnp.inf, lt)
    l2 = jnp.max(ltm, axis=0, keepdims=True)
    i2 = jnp.min(jnp.where(ltm == l2, iota_e, E), axis=0, keepdims=True)
    m1 = (iota_e == i2)
    w0 = 1.0 / (1.0 + jnp.exp(l2 - l1))
    w1v = 1.0 - w0

    m = jnp.concatenate([m0, m1], axis=1).astype(jnp.float32)
    tri = (lax.broadcasted_iota(jnp.int32, (128, 128), 0)
           <= lax.broadcasted_iota(jnp.int32, (128, 128), 1)).astype(jnp.float32)
    chunks = []
    run = jnp.zeros((E, 1), jnp.float32)
    for c in range(P // 128):
        mc = lax.slice(m, (0, c * 128), (E, (c + 1) * 128))
        local = lax.dot_general(mc, tri, (((1,), (0,)), ((), ())),
                                preferred_element_type=jnp.float32)
        chunks.append(local + run)
        run = run + lax.slice(local, (0, 127), (E, 128))
    cum = jnp.concatenate(chunks, axis=1)
    cnt = run
    pcnt = jnp.ceil(cnt * (1.0 / BM)) * BM
    low = (lax.broadcasted_iota(jnp.int32, (E, E), 0)
           > lax.broadcasted_iota(jnp.int32, (E, E), 1)).astype(jnp.float32)
    offs = lax.dot_general(low, pcnt, (((1,), (0,)), ((), ())),
                           preferred_element_type=jnp.float32)

    m0f = m0.astype(jnp.float32)
    m1f = m1.astype(jnp.float32)
    c0 = jnp.sum(m0f * lax.slice(cum, (0, 0), (E, T)), axis=0, keepdims=True)
    c1 = jnp.sum(m1f * lax.slice(cum, (0, T), (E, 2 * T)), axis=0, keepdims=True)
    o0 = jnp.sum(m0f * offs, axis=0, keepdims=True)
    o1 = jnp.sum(m1f * offs, axis=0, keepdims=True)
    pos0 = o0 + c0 - 1.0
    pos1 = o1 + c1 - 1.0
    pos_ref[...] = jnp.concatenate([pos0, pos1], axis=0).astype(jnp.int32)
    topw_ref[...] = jnp.concatenate([w0, w1v], axis=0)

    ends = offs + pcnt
    qs = lax.broadcasted_iota(jnp.int32, (1, NB), 1).astype(jnp.float32) * BM
    bexp = jnp.sum((ends <= qs).astype(jnp.float32), axis=0, keepdims=True)
    bexp = jnp.minimum(bexp, float(E - 1))
    nact = (jnp.sum(pcnt) * (1.0 / BM)).reshape(1, 1)
    meta_ref[...] = jnp.concatenate([nact, bexp], axis=1).astype(jnp.int32)


def _ffn_body(meta_ref, xg_ref, w1_ref, v1_ref, w2_ref, y_ref):
    b = pl.program_id(0)

    @pl.when(b < meta_ref[0])
    def _():
        xb = xg_ref[...]
        a = lax.dot_general(xb, w1_ref[0], (((1,), (1,)), ((), ())),
                            preferred_element_type=jnp.float32)
        u = lax.dot_general(xb, v1_ref[0], (((1,), (1,)), ((), ())),
                            preferred_element_type=jnp.float32)
        h = (a * jax.nn.sigmoid(a)) * u
        y_ref[...] = jnp.dot(h, w2_ref[0], preferred_element_type=jnp.float32)


def _sc_dispatch_body(x_hbm, pos_hbm, xg_hbm, idx_v, rows_v, sem):
    wid = lax.axis_index("s") * 2 + lax.axis_index("c")
    per_w = P // NW
    base = wid * per_w

    def chunk(c, carry):
        p0 = base + c * G_CH
        t0 = p0 - (p0 // T) * T
        pltpu.sync_copy(pos_hbm.at[pl.ds(p0, G_CH)], idx_v)
        pltpu.sync_copy(x_hbm.at[pl.ds(t0, G_CH)], rows_v)
        pltpu.async_copy(rows_v, xg_hbm.at[idx_v], sem).wait()
        return carry

    lax.fori_loop(0, per_w // G_CH, chunk, 0)


def _sc_combine_body(y_hbm, pos_hbm, w_hbm, out_hbm,
                     i0_v, i1_v, r0_v, r1_v, o_v, w0_v, w1_v, sem):
    wid = lax.axis_index("s") * 2 + lax.axis_index("c")
    per_w = T // NW
    base = wid * per_w

    def chunk(c, carry):
        t0 = base + c * C_CH
        pltpu.sync_copy(pos_hbm.at[pl.ds(t0, C_CH)], i0_v)
        pltpu.sync_copy(pos_hbm.at[pl.ds(T + t0, C_CH)], i1_v)
        pltpu.sync_copy(w_hbm.at[pl.ds(t0, C_CH)], w0_v.at[pl.ds(0, C_CH)])
        pltpu.sync_copy(w_hbm.at[pl.ds(T + t0, C_CH)], w1_v.at[pl.ds(0, C_CH)])
        pltpu.async_copy(y_hbm.at[i0_v], r0_v, sem).wait()
        pltpu.async_copy(y_hbm.at[i1_v], r1_v, sem).wait()

        def row(i, rcarry):
            a = w0_v[pl.ds(i, 16)][0]
            b = w1_v[pl.ds(i, 16)][0]
            for j in range(D // 16):
                sl = pl.ds(j * 16, 16)
                o_v[i, sl] = a * r0_v[i, sl] + b * r1_v[i, sl]
            return rcarry

        lax.fori_loop(0, C_CH, row, 0)
        pltpu.sync_copy(o_v, out_hbm.at[pl.ds(t0, C_CH)])
        return carry

    lax.fori_loop(0, per_w // C_CH, chunk, 0)


@functools.cache
def _get_sc_kernels():
    mesh = plsc.VectorSubcoreMesh(core_axis_name="c", subcore_axis_name="s")
    dispatch = pl.kernel(
        _sc_dispatch_body,
        out_type=jax.ShapeDtypeStruct((NQ, D), jnp.float32),
        mesh=mesh,
        scratch_types=[
            pltpu.VMEM((G_CH,), jnp.int32),
            pltpu.VMEM((G_CH, D), jnp.float32),
            pltpu.SemaphoreType.DMA,
        ],
    )
    combine = pl.kernel(
        _sc_combine_body,
        out_type=jax.ShapeDtypeStruct((T, D), jnp.float32),
        mesh=mesh,
        scratch_types=[
            pltpu.VMEM((C_CH,), jnp.int32),
            pltpu.VMEM((C_CH,), jnp.int32),
            pltpu.VMEM((C_CH, D), jnp.float32),
            pltpu.VMEM((C_CH, D), jnp.float32),
            pltpu.VMEM((C_CH, D), jnp.float32),
            pltpu.VMEM((C_CH + 16,), jnp.float32),
            pltpu.VMEM((C_CH + 16,), jnp.float32),
            pltpu.SemaphoreType.DMA,
        ],
    )
    return dispatch, combine

_router_sched = pl.pallas_call(
    _router_sched_body,
    out_shape=[
        jax.ShapeDtypeStruct((TK, T), jnp.int32),
        jax.ShapeDtypeStruct((TK, T), jnp.float32),
        jax.ShapeDtypeStruct((1, NB + 1), jnp.int32),
    ],
)

_ffn = pl.pallas_call(
    _ffn_body,
    grid_spec=pltpu.PrefetchScalarGridSpec(
        num_scalar_prefetch=1,
        grid=(NB,),
        in_specs=[
            pl.BlockSpec((BM, D), lambda b, m: (b, 0)),
            pl.BlockSpec((1, F, D), lambda b, m: (m[b + 1], 0, 0)),
            pl.BlockSpec((1, F, D), lambda b, m: (m[b + 1], 0, 0)),
            pl.BlockSpec((1, F, D), lambda b, m: (m[b + 1], 0, 0)),
        ],
        out_specs=pl.BlockSpec((BM, D), lambda b, m: (b, 0)),
    ),
    out_shape=jax.ShapeDtypeStruct((NQ, D), jnp.float32),
)


@jax.jit
def kernel(hidden_states, router_w, w1, v1, w2):
    B, S, Dh = hidden_states.shape
    x = hidden_states.reshape(T, D)
    dispatch, combine = _get_sc_kernels()
    pos2, topw2, meta2 = _router_sched(x, router_w)
    posf = pos2.reshape(P)
    wf = topw2.reshape(P)
    meta = meta2.reshape(NB + 1)
    xg = dispatch(x, posf)
    y = _ffn(meta, xg, w1, v1, w2)
    out = combine(y, posf, wf)
    return out.reshape(B, S, Dh)
